# Initial kernel scaffold; baseline (speedup 1.0000x reference)
import jax
import jax.numpy as jnp
from jax.experimental import pallas as pl


def kernel(features, prev_features, neighbor_idx, W, b):
    raise NotImplementedError("write your pallas kernel here")


# speedup vs baseline: 1.4209x; 1.4209x over previous
import functools

import jax
import jax.numpy as jnp
from jax import lax
from jax.experimental import pallas as pl
from jax.experimental.pallas import tpu as pltpu
from jax.experimental.pallas import tpu_sc as plsc

N = 100000
D = 128
OUT = 128
K = 4

NC = 2
NS = 16
NW = NC * NS

NTOT = 100352
RPW = NTOT // NW
C = 56
G = RPW // C

BT = 2048
assert NTOT % BT == 0


def _tc_body(feat_ref, prev_ref, w0t_ref, wb_ref, b_ref, a_ref, p_ref):
    a_ref[...] = (
        jnp.dot(feat_ref[...], w0t_ref[...], preferred_element_type=jnp.float32)
        + b_ref[0][None, :]
    )
    p_ref[...] = jnp.dot(prev_ref[...], wb_ref[...], preferred_element_type=jnp.float32)


def _tc_phase(feat_p, prev_p, w0t, wb, b8):
    grid = (NTOT // BT,)
    return pl.pallas_call(
        _tc_body,
        grid=grid,
        in_specs=[
            pl.BlockSpec((BT, D), lambda i: (i, 0)),
            pl.BlockSpec((BT, D), lambda i: (i, 0)),
            pl.BlockSpec((D, OUT), lambda i: (0, 0)),
            pl.BlockSpec((D, K * OUT), lambda i: (0, 0)),
            pl.BlockSpec((8, OUT), lambda i: (0, 0)),
        ],
        out_specs=[
            pl.BlockSpec((BT, OUT), lambda i: (i, 0)),
            pl.BlockSpec((BT, K * OUT), lambda i: (i, 0)),
        ],
        out_shape=[
            jax.ShapeDtypeStruct((NTOT, OUT), jnp.float32),
            jax.ShapeDtypeStruct((NTOT, K * OUT), jnp.float32),
        ],
    )(feat_p, prev_p, w0t, wb, b8)


def _sc_gather_sum(p_flat, a_full, idx_flat):
    mesh = plsc.VectorSubcoreMesh(core_axis_name="c", subcore_axis_name="s")

    @functools.partial(
        pl.kernel,
        out_type=jax.ShapeDtypeStruct((NTOT, OUT), jnp.float32),
        mesh=mesh,
        scratch_types=[
            pltpu.VMEM((K * C,), jnp.int32),
            pltpu.VMEM((K * C,), jnp.int32),
            pltpu.VMEM((K * C, OUT), jnp.float32),
            pltpu.VMEM((K * C, OUT), jnp.float32),
            pltpu.VMEM((C, OUT), jnp.float32),
            pltpu.VMEM((C, OUT), jnp.float32),
            pltpu.SemaphoreType.DMA,
            pltpu.SemaphoreType.DMA,
        ],
    )
    def sc_kernel(p_hbm, a_hbm, idx_hbm, out_hbm,
                  fidx0, fidx1, gb0, gb1, av, ov, sem0, sem1):
        cid = lax.axis_index("c")
        sid = lax.axis_index("s")
        wid = sid * NC + cid
        base = wid * RPW
        kpat = lax.rem(lax.iota(jnp.int32, 16), 4)

        def start(g, fidx, gb, sem):
            off = (base + g * C) * K
            pltpu.sync_copy(idx_hbm.at[pl.ds(off, K * C)], fidx)
            for v in range(K * C // 16):
                sl = pl.ds(v * 16, 16)
                fidx[sl] = fidx[sl] * 4 + kpat
            pltpu.async_copy(p_hbm.at[fidx], gb, sem)

        def finish(g, fidx, gb, sem):
            pltpu.make_async_copy(p_hbm.at[fidx], gb, sem).wait()
            row0 = base + g * C
            pltpu.sync_copy(a_hbm.at[pl.ds(row0, C)], av)

            def row(c, carry):
                for r in range(OUT // 16):
                    sl = pl.ds(r * 16, 16)
                    acc = av[c, sl]
                    acc = acc + gb[4 * c, sl]
                    acc = acc + gb[4 * c + 1, sl]
                    acc = acc + gb[4 * c + 2, sl]
                    acc = acc + gb[4 * c + 3, sl]
                    ov[c, sl] = acc
                return carry

            lax.fori_loop(0, C, row, 0)
            pltpu.sync_copy(ov, out_hbm.at[pl.ds(row0, C)])

        start(0, fidx0, gb0, sem0)

        def pair(p, carry):
            g0 = p * 2
            start(g0 + 1, fidx1, gb1, sem1)
            finish(g0, fidx0, gb0, sem0)

            @pl.when(g0 + 2 < G)
            def _():
                start(g0 + 2, fidx0, gb0, sem0)

            finish(g0 + 1, fidx1, gb1, sem1)
            return carry

        lax.fori_loop(0, G // 2, pair, 0)

    return sc_kernel(p_flat, a_full, idx_flat)


def kernel(features, prev_features, neighbor_idx, W, b):
    pad = NTOT - N
    feat_p = jnp.pad(features, ((0, pad), (0, 0)))
    prev_p = jnp.pad(prev_features, ((0, pad), (0, 0)))
    idx_p = jnp.pad(neighbor_idx.astype(jnp.int32), ((0, pad), (0, 0)))
    idx_flat = idx_p.reshape(NTOT * K)

    w0t = W[:, :D].T
    wb = W[:, D:].reshape(OUT, K, D).transpose(2, 1, 0).reshape(D, K * OUT)
    b8 = jnp.broadcast_to(b[None, :], (8, OUT))

    a_full, p_blk = _tc_phase(feat_p, prev_p, w0t, wb, b8)
    p_flat = p_blk.reshape(NTOT * K, OUT)

    out_full = _sc_gather_sum(p_flat, a_full, idx_flat)
    return out_full[:N]


# speedup vs baseline: 1.6304x; 1.1475x over previous
import functools

import jax
import jax.numpy as jnp
from jax import lax
from jax.experimental import pallas as pl
from jax.experimental.pallas import tpu as pltpu
from jax.experimental.pallas import tpu_sc as plsc

N = 100000
D = 128
OUT = 128
K = 4

NC = 2
NS = 16
NW = NC * NS

NTOT = 100352
RPW = NTOT // NW
C = 56
G = RPW // C

BT = 2048
assert NTOT % BT == 0


def _tc_body(feat_ref, prev_ref, w0t_ref, wb_ref, b_ref, a_ref, p_ref):
    a_ref[...] = (
        jnp.dot(feat_ref[...], w0t_ref[...], preferred_element_type=jnp.float32)
        + b_ref[0][None, :]
    )
    p_ref[...] = jnp.dot(prev_ref[...], wb_ref[...], preferred_element_type=jnp.float32)


def _tc_phase(feat_p, prev_p, w0t, wb, b8):
    grid = (NTOT // BT,)
    return pl.pallas_call(
        _tc_body,
        grid=grid,
        in_specs=[
            pl.BlockSpec((BT, D), lambda i: (i, 0)),
            pl.BlockSpec((BT, D), lambda i: (i, 0)),
            pl.BlockSpec((D, OUT), lambda i: (0, 0)),
            pl.BlockSpec((D, K * OUT), lambda i: (0, 0)),
            pl.BlockSpec((8, OUT), lambda i: (0, 0)),
        ],
        out_specs=[
            pl.BlockSpec((BT, OUT), lambda i: (i, 0)),
            pl.BlockSpec((BT, K * OUT), lambda i: (i, 0)),
        ],
        out_shape=[
            jax.ShapeDtypeStruct((NTOT, OUT), jnp.float32),
            jax.ShapeDtypeStruct((NTOT, K * OUT), jnp.float32),
        ],
    )(feat_p, prev_p, w0t, wb, b8)


def _sc_gather_sum(p_flat, a_full, idx_flat):
    mesh = plsc.VectorSubcoreMesh(core_axis_name="c", subcore_axis_name="s")

    @functools.partial(
        pl.kernel,
        out_type=jax.ShapeDtypeStruct((NTOT, OUT), jnp.float32),
        mesh=mesh,
        scratch_types=[
            pltpu.VMEM((RPW * K,), jnp.int32),
            pltpu.VMEM((K * C, OUT), jnp.float32),
            pltpu.VMEM((K * C, OUT), jnp.float32),
            pltpu.VMEM((C, OUT), jnp.float32),
            pltpu.VMEM((C, OUT), jnp.float32),
            pltpu.VMEM((C, OUT), jnp.float32),
            pltpu.VMEM((C, OUT), jnp.float32),
            pltpu.SemaphoreType.DMA,
            pltpu.SemaphoreType.DMA,
            pltpu.SemaphoreType.DMA,
            pltpu.SemaphoreType.DMA,
            pltpu.SemaphoreType.DMA,
            pltpu.SemaphoreType.DMA,
        ],
    )
    def sc_kernel(p_hbm, a_hbm, idx_hbm, out_hbm,
                  idxall, gb0, gb1, av0, av1, ov0, ov1,
                  sg0, sg1, sa0, sa1, so0, so1):
        cid = lax.axis_index("c")
        sid = lax.axis_index("s")
        wid = sid * NC + cid
        base = wid * RPW
        kpat = lax.rem(lax.iota(jnp.int32, 16), 4)

        pltpu.sync_copy(idx_hbm.at[pl.ds(base * K, RPW * K)], idxall)

        def conv(v, carry):
            sl = pl.ds(v * 16, 16)
            idxall[sl] = idxall[sl] * 4 + kpat
            return carry

        lax.fori_loop(0, RPW * K // 16, conv, 0)

        def start(g, gb, av, sg, sa):
            pltpu.async_copy(
                p_hbm.at[idxall.at[pl.ds(g * K * C, K * C)]], gb, sg)
            pltpu.async_copy(a_hbm.at[pl.ds(base + g * C, C)], av, sa)

        def finish(g, gb, av, ov, sg, sa, so):
            pltpu.make_async_copy(
                p_hbm.at[idxall.at[pl.ds(g * K * C, K * C)]], gb, sg).wait()
            pltpu.make_async_copy(
                a_hbm.at[pl.ds(base + g * C, C)], av, sa).wait()

            @pl.when(g >= 2)
            def _():
                pltpu.make_async_copy(
                    ov, out_hbm.at[pl.ds(base, C)], so).wait()

            def row(c, carry):
                for r in range(OUT // 16):
                    sl = pl.ds(r * 16, 16)
                    acc = av[c, sl]
                    acc = acc + gb[4 * c, sl]
                    acc = acc + gb[4 * c + 1, sl]
                    acc = acc + gb[4 * c + 2, sl]
                    acc = acc + gb[4 * c + 3, sl]
                    ov[c, sl] = acc
                return carry

            lax.fori_loop(0, C, row, 0)
            pltpu.async_copy(ov, out_hbm.at[pl.ds(base + g * C, C)], so)

        start(0, gb0, av0, sg0, sa0)

        def pair(p, carry):
            g0 = p * 2
            start(g0 + 1, gb1, av1, sg1, sa1)
            finish(g0, gb0, av0, ov0, sg0, sa0, so0)

            @pl.when(g0 + 2 < G)
            def _():
                start(g0 + 2, gb0, av0, sg0, sa0)

            finish(g0 + 1, gb1, av1, ov1, sg1, sa1, so1)
            return carry

        lax.fori_loop(0, G // 2, pair, 0)

        pltpu.make_async_copy(ov0, out_hbm.at[pl.ds(base, C)], so0).wait()
        pltpu.make_async_copy(ov1, out_hbm.at[pl.ds(base, C)], so1).wait()

    return sc_kernel(p_flat, a_full, idx_flat)


def kernel(features, prev_features, neighbor_idx, W, b):
    pad = NTOT - N
    feat_p = jnp.pad(features, ((0, pad), (0, 0)))
    prev_p = jnp.pad(prev_features, ((0, pad), (0, 0)))
    idx_p = jnp.pad(neighbor_idx.astype(jnp.int32), ((0, pad), (0, 0)))
    idx_flat = idx_p.reshape(NTOT * K)

    w0t = W[:, :D].T
    wb = W[:, D:].reshape(OUT, K, D).transpose(2, 1, 0).reshape(D, K * OUT)
    b8 = jnp.broadcast_to(b[None, :], (8, OUT))

    a_full, p_blk = _tc_phase(feat_p, prev_p, w0t, wb, b8)
    p_flat = p_blk.reshape(NTOT * K, OUT)

    out_full = _sc_gather_sum(p_flat, a_full, idx_flat)
    return out_full[:N]
